# baseline (speedup 1.0000x reference)
import jax
import jax.numpy as jnp
from jax import lax
from jax.experimental import pallas as pl
from jax.experimental.pallas import tpu as pltpu
from jax.experimental.pallas import tpu_sc as plsc

N_NODES = 10000
D = 128
N_PAD = 10240
NW = 32
CHUNK = 64
IDX_BLK = 16
ROWS_PER_TILE = N_PAD // 16


def _tc_layer_body(a0_ref, a1_ref, c0_ref, c1_ref, x_ref, w_ref, r_ref, b_ref,
                   o_ref):
    c = jnp.maximum(c0_ref[:, 0:1] + c1_ref[:, 0:1], 1.0)
    agg = (a0_ref[...] + a1_ref[...]) / c
    o_ref[...] = jax.nn.relu(
        jnp.dot(agg, w_ref[...], preferred_element_type=jnp.float32)
        + jnp.dot(x_ref[...], r_ref[...], preferred_element_type=jnp.float32)
        + b_ref[...])


def _tc_layer(a0, a1, c0, c1, x, w, r, b, block=1024):
    n = x.shape[0]
    grid = (n // block,)
    return pl.pallas_call(
        _tc_layer_body,
        grid=grid,
        in_specs=[
            pl.BlockSpec((block, D), lambda i: (i, 0)),
            pl.BlockSpec((block, D), lambda i: (i, 0)),
            pl.BlockSpec((block, D), lambda i: (i, 0)),
            pl.BlockSpec((block, D), lambda i: (i, 0)),
            pl.BlockSpec((block, D), lambda i: (i, 0)),
            pl.BlockSpec((D, D), lambda i: (0, 0)),
            pl.BlockSpec((D, D), lambda i: (0, 0)),
            pl.BlockSpec((1, D), lambda i: (0, 0)),
        ],
        out_specs=pl.BlockSpec((block, D), lambda i: (i, 0)),
        out_shape=jax.ShapeDtypeStruct((n, D), jnp.float32),
    )(a0, a1, c0, c1, x, w, r, b)


def _seg_step_pipelined(y_hbm, acc_sh, src_v, dst_v, bufs, gsem, ssem):
    nbuf = len(bufs)
    depth = nbuf - 1
    g = {}
    for k in range(min(depth, IDX_BLK)):
        g[k] = pltpu.async_copy(y_hbm.at[src_v.at[k]], bufs[k % nbuf], gsem)
    scat = {}
    waited = 0
    for j in range(IDX_BLK):
        g[j].wait()
        scat[j] = pltpu.async_copy(bufs[j % nbuf], acc_sh.at[dst_v.at[j]],
                                   ssem, add=True)
        k = j + depth
        if k < IDX_BLK:
            while waited <= k - nbuf:
                scat[waited].wait()
                waited += 1
            g[k] = pltpu.async_copy(y_hbm.at[src_v.at[k]], bufs[k % nbuf], gsem)
    for w in range(waited, IDX_BLK):
        scat[w].wait()


def _sc_segsum_counts_body(y_hbm, src_hbm, dst_hbm, z128_hbm, ones_hbm,
                           acc_out, cnt_out,
                           acc_sh, src_v, dst_v, buf0, buf1, buf2, buf3,
                           gsem, ssem):
    sid = lax.axis_index("s")
    cid = lax.axis_index("c")
    g = cid * 16 + sid
    n_chunks = src_hbm.shape[1]
    row0 = sid * ROWS_PER_TILE

    def zero_my_slice():
        pltpu.sync_copy(z128_hbm, buf0)
        for t in range(ROWS_PER_TILE // CHUNK):
            pltpu.sync_copy(buf0, acc_sh.at[pl.ds(row0 + t * CHUNK, CHUNK)])

    zero_my_slice()
    pltpu.sync_copy(ones_hbm, buf0)
    plsc.subcore_barrier()

    def step_cnt(blk, carry):
        pltpu.sync_copy(dst_hbm.at[g, pl.ds(blk * IDX_BLK, IDX_BLK)], dst_v)
        descs = [pltpu.async_copy(buf0, acc_sh.at[dst_v.at[j]], ssem, add=True)
                 for j in range(IDX_BLK)]
        for d in descs:
            d.wait()
        return carry

    lax.fori_loop(0, n_chunks // IDX_BLK, step_cnt, 0)
    plsc.subcore_barrier()
    pltpu.sync_copy(acc_sh.at[pl.ds(row0, ROWS_PER_TILE)],
                    cnt_out.at[cid, pl.ds(row0, ROWS_PER_TILE)])

    zero_my_slice()
    plsc.subcore_barrier()

    def step(blk, carry):
        pltpu.sync_copy(src_hbm.at[g, pl.ds(blk * IDX_BLK, IDX_BLK)], src_v)
        pltpu.sync_copy(dst_hbm.at[g, pl.ds(blk * IDX_BLK, IDX_BLK)], dst_v)
        _seg_step_pipelined(y_hbm, acc_sh, src_v, dst_v,
                            (buf0, buf1, buf2, buf3), gsem, ssem)
        return carry

    lax.fori_loop(0, n_chunks // IDX_BLK, step, 0)
    plsc.subcore_barrier()
    pltpu.sync_copy(acc_sh.at[pl.ds(row0, ROWS_PER_TILE)],
                    acc_out.at[cid, pl.ds(row0, ROWS_PER_TILE)])


def _sc_segsum_body(y_hbm, src_hbm, dst_hbm, z128_hbm,
                    acc_out,
                    acc_sh, src_v, dst_v, buf0, buf1, buf2, buf3, gsem, ssem):
    sid = lax.axis_index("s")
    cid = lax.axis_index("c")
    g = cid * 16 + sid
    n_chunks = src_hbm.shape[1]

    row0 = sid * ROWS_PER_TILE
    pltpu.sync_copy(z128_hbm, buf0)
    for t in range(ROWS_PER_TILE // CHUNK):
        pltpu.sync_copy(buf0, acc_sh.at[pl.ds(row0 + t * CHUNK, CHUNK)])
    plsc.subcore_barrier()

    def step(blk, carry):
        pltpu.sync_copy(src_hbm.at[g, pl.ds(blk * IDX_BLK, IDX_BLK)], src_v)
        pltpu.sync_copy(dst_hbm.at[g, pl.ds(blk * IDX_BLK, IDX_BLK)], dst_v)
        _seg_step_pipelined(y_hbm, acc_sh, src_v, dst_v,
                            (buf0, buf1, buf2, buf3), gsem, ssem)
        return carry

    lax.fori_loop(0, n_chunks // IDX_BLK, step, 0)
    plsc.subcore_barrier()

    row0 = sid * ROWS_PER_TILE
    pltpu.sync_copy(acc_sh.at[pl.ds(row0, ROWS_PER_TILE)],
                    acc_out.at[cid, pl.ds(row0, ROWS_PER_TILE)])


_SC_MESH = plsc.VectorSubcoreMesh(core_axis_name="c", subcore_axis_name="s")


def _sc_segsum_counts(y, src_r, dst_r, z128, ones128):
    return pl.kernel(
        _sc_segsum_counts_body,
        out_type=[
            jax.ShapeDtypeStruct((2, N_PAD, D), jnp.float32),
            jax.ShapeDtypeStruct((2, N_PAD, D), jnp.float32),
        ],
        mesh=_SC_MESH,
        scratch_types=[
            pltpu.VMEM_SHARED((N_PAD, D), jnp.float32),
            pltpu.VMEM((IDX_BLK, CHUNK), jnp.int32),
            pltpu.VMEM((IDX_BLK, CHUNK), jnp.int32),
            pltpu.VMEM((CHUNK, D), jnp.float32),
            pltpu.VMEM((CHUNK, D), jnp.float32),
            pltpu.VMEM((CHUNK, D), jnp.float32),
            pltpu.VMEM((CHUNK, D), jnp.float32),
            pltpu.SemaphoreType.DMA,
            pltpu.SemaphoreType.DMA,
        ],
    )(y, src_r, dst_r, z128, ones128)


def _sc_segsum(y, src_r, dst_r, z128):
    n_chunks = src_r.shape[1]
    return pl.kernel(
        _sc_segsum_body,
        out_type=jax.ShapeDtypeStruct((2, N_PAD, D), jnp.float32),
        mesh=_SC_MESH,
        scratch_types=[
            pltpu.VMEM_SHARED((N_PAD, D), jnp.float32),
            pltpu.VMEM((IDX_BLK, CHUNK), jnp.int32),
            pltpu.VMEM((IDX_BLK, CHUNK), jnp.int32),
            pltpu.VMEM((CHUNK, D), jnp.float32),
            pltpu.VMEM((CHUNK, D), jnp.float32),
            pltpu.VMEM((CHUNK, D), jnp.float32),
            pltpu.VMEM((CHUNK, D), jnp.float32),
            pltpu.SemaphoreType.DMA,
            pltpu.SemaphoreType.DMA,
        ],
    )(y, src_r, dst_r, z128)


def kernel(x, edge_index, W1, R1, b1, W2, R2, b2):
    n_edges = edge_index.shape[1]
    gran = NW * CHUNK * IDX_BLK
    e_pad = ((n_edges + gran - 1) // gran) * gran
    per_tile = e_pad // NW

    src = edge_index[0].astype(jnp.int32)
    dst = edge_index[1].astype(jnp.int32)
    pad_idx = jnp.full((e_pad - n_edges,), N_NODES, dtype=jnp.int32)
    src_r = jnp.concatenate([src, pad_idx]).reshape(NW, per_tile // CHUNK, CHUNK)
    dst_r = jnp.concatenate([dst, pad_idx]).reshape(NW, per_tile // CHUNK, CHUNK)

    x_p = jnp.pad(x, ((0, N_PAD - x.shape[0]), (0, 0)))
    b1_p = b1.reshape(1, D)
    b2_p = b2.reshape(1, D)

    z128 = jnp.zeros((CHUNK, D), jnp.float32)
    ones128 = jnp.ones((CHUNK, D), jnp.float32)

    acc1, cnt = _sc_segsum_counts(x_p, src_r, dst_r, z128, ones128)
    h = _tc_layer(acc1[0], acc1[1], cnt[0], cnt[1], x_p, W1, R1, b1_p)
    acc2 = _sc_segsum(h, src_r, dst_r, z128)
    out = _tc_layer(acc2[0], acc2[1], cnt[0], cnt[1], h, W2, R2, b2_p)
    return out[:N_NODES]


# speedup vs baseline: 1.0598x; 1.0598x over previous
import jax
import jax.numpy as jnp
from jax import lax
from jax.experimental import pallas as pl
from jax.experimental.pallas import tpu as pltpu
from jax.experimental.pallas import tpu_sc as plsc

N_NODES = 10000
D = 128
N_PAD = 10240
NW = 32
CHUNK = 128
IDX_BLK = 16
ROWS_PER_TILE = N_PAD // 16


def _tc_layer_body(a0_ref, a1_ref, c0_ref, c1_ref, x_ref, w_ref, r_ref, b_ref,
                   o_ref):
    c = jnp.maximum(c0_ref[:, 0:1] + c1_ref[:, 0:1], 1.0)
    agg = (a0_ref[...] + a1_ref[...]) / c
    o_ref[...] = jax.nn.relu(
        jnp.dot(agg, w_ref[...], preferred_element_type=jnp.float32)
        + jnp.dot(x_ref[...], r_ref[...], preferred_element_type=jnp.float32)
        + b_ref[...])


def _tc_layer(a0, a1, c0, c1, x, w, r, b, block=1024):
    n = x.shape[0]
    grid = (n // block,)
    return pl.pallas_call(
        _tc_layer_body,
        grid=grid,
        in_specs=[
            pl.BlockSpec((block, D), lambda i: (i, 0)),
            pl.BlockSpec((block, D), lambda i: (i, 0)),
            pl.BlockSpec((block, D), lambda i: (i, 0)),
            pl.BlockSpec((block, D), lambda i: (i, 0)),
            pl.BlockSpec((block, D), lambda i: (i, 0)),
            pl.BlockSpec((D, D), lambda i: (0, 0)),
            pl.BlockSpec((D, D), lambda i: (0, 0)),
            pl.BlockSpec((1, D), lambda i: (0, 0)),
        ],
        out_specs=pl.BlockSpec((block, D), lambda i: (i, 0)),
        out_shape=jax.ShapeDtypeStruct((n, D), jnp.float32),
    )(a0, a1, c0, c1, x, w, r, b)


def _seg_step_pipelined(y_hbm, acc_sh, src_v, dst_v, bufs, gsem, ssem):
    nbuf = len(bufs)
    depth = nbuf - 1
    g = {}
    for k in range(min(depth, IDX_BLK)):
        g[k] = pltpu.async_copy(y_hbm.at[src_v.at[k]], bufs[k % nbuf], gsem)
    scat = {}
    waited = 0
    for j in range(IDX_BLK):
        g[j].wait()
        scat[j] = pltpu.async_copy(bufs[j % nbuf], acc_sh.at[dst_v.at[j]],
                                   ssem, add=True)
        k = j + depth
        if k < IDX_BLK:
            while waited <= k - nbuf:
                scat[waited].wait()
                waited += 1
            g[k] = pltpu.async_copy(y_hbm.at[src_v.at[k]], bufs[k % nbuf], gsem)
    for w in range(waited, IDX_BLK):
        scat[w].wait()


def _sc_segsum_counts_body(y_hbm, src_hbm, dst_hbm, z128_hbm, ones_hbm,
                           acc_out, cnt_out,
                           acc_sh, src_v, dst_v, buf0, buf1, gsem, ssem):
    sid = lax.axis_index("s")
    cid = lax.axis_index("c")
    g = cid * 16 + sid
    n_chunks = src_hbm.shape[1]
    row0 = sid * ROWS_PER_TILE

    def zero_my_slice():
        pltpu.sync_copy(z128_hbm, buf0)
        for t in range(ROWS_PER_TILE // CHUNK):
            pltpu.sync_copy(buf0, acc_sh.at[pl.ds(row0 + t * CHUNK, CHUNK)])

    zero_my_slice()
    pltpu.sync_copy(ones_hbm, buf0)
    plsc.subcore_barrier()

    def step_cnt(blk, carry):
        pltpu.sync_copy(dst_hbm.at[g, pl.ds(blk * IDX_BLK, IDX_BLK)], dst_v)
        descs = [pltpu.async_copy(buf0, acc_sh.at[dst_v.at[j]], ssem, add=True)
                 for j in range(IDX_BLK)]
        for d in descs:
            d.wait()
        return carry

    lax.fori_loop(0, n_chunks // IDX_BLK, step_cnt, 0)
    plsc.subcore_barrier()
    pltpu.sync_copy(acc_sh.at[pl.ds(row0, ROWS_PER_TILE)],
                    cnt_out.at[cid, pl.ds(row0, ROWS_PER_TILE)])

    zero_my_slice()
    plsc.subcore_barrier()

    def step(blk, carry):
        pltpu.sync_copy(src_hbm.at[g, pl.ds(blk * IDX_BLK, IDX_BLK)], src_v)
        pltpu.sync_copy(dst_hbm.at[g, pl.ds(blk * IDX_BLK, IDX_BLK)], dst_v)
        _seg_step_pipelined(y_hbm, acc_sh, src_v, dst_v,
                            (buf0, buf1), gsem, ssem)
        return carry

    lax.fori_loop(0, n_chunks // IDX_BLK, step, 0)
    plsc.subcore_barrier()
    pltpu.sync_copy(acc_sh.at[pl.ds(row0, ROWS_PER_TILE)],
                    acc_out.at[cid, pl.ds(row0, ROWS_PER_TILE)])


def _sc_segsum_body(y_hbm, src_hbm, dst_hbm, z128_hbm,
                    acc_out,
                    acc_sh, src_v, dst_v, buf0, buf1, gsem, ssem):
    sid = lax.axis_index("s")
    cid = lax.axis_index("c")
    g = cid * 16 + sid
    n_chunks = src_hbm.shape[1]

    row0 = sid * ROWS_PER_TILE
    pltpu.sync_copy(z128_hbm, buf0)
    for t in range(ROWS_PER_TILE // CHUNK):
        pltpu.sync_copy(buf0, acc_sh.at[pl.ds(row0 + t * CHUNK, CHUNK)])
    plsc.subcore_barrier()

    def step(blk, carry):
        pltpu.sync_copy(src_hbm.at[g, pl.ds(blk * IDX_BLK, IDX_BLK)], src_v)
        pltpu.sync_copy(dst_hbm.at[g, pl.ds(blk * IDX_BLK, IDX_BLK)], dst_v)
        _seg_step_pipelined(y_hbm, acc_sh, src_v, dst_v,
                            (buf0, buf1), gsem, ssem)
        return carry

    lax.fori_loop(0, n_chunks // IDX_BLK, step, 0)
    plsc.subcore_barrier()

    row0 = sid * ROWS_PER_TILE
    pltpu.sync_copy(acc_sh.at[pl.ds(row0, ROWS_PER_TILE)],
                    acc_out.at[cid, pl.ds(row0, ROWS_PER_TILE)])


_SC_MESH = plsc.VectorSubcoreMesh(core_axis_name="c", subcore_axis_name="s")


def _sc_segsum_counts(y, src_r, dst_r, z128, ones128):
    return pl.kernel(
        _sc_segsum_counts_body,
        out_type=[
            jax.ShapeDtypeStruct((2, N_PAD, D), jnp.float32),
            jax.ShapeDtypeStruct((2, N_PAD, D), jnp.float32),
        ],
        mesh=_SC_MESH,
        scratch_types=[
            pltpu.VMEM_SHARED((N_PAD, D), jnp.float32),
            pltpu.VMEM((IDX_BLK, CHUNK), jnp.int32),
            pltpu.VMEM((IDX_BLK, CHUNK), jnp.int32),
            pltpu.VMEM((CHUNK, D), jnp.float32),
            pltpu.VMEM((CHUNK, D), jnp.float32),
            pltpu.SemaphoreType.DMA,
            pltpu.SemaphoreType.DMA,
        ],
    )(y, src_r, dst_r, z128, ones128)


def _sc_segsum(y, src_r, dst_r, z128):
    n_chunks = src_r.shape[1]
    return pl.kernel(
        _sc_segsum_body,
        out_type=jax.ShapeDtypeStruct((2, N_PAD, D), jnp.float32),
        mesh=_SC_MESH,
        scratch_types=[
            pltpu.VMEM_SHARED((N_PAD, D), jnp.float32),
            pltpu.VMEM((IDX_BLK, CHUNK), jnp.int32),
            pltpu.VMEM((IDX_BLK, CHUNK), jnp.int32),
            pltpu.VMEM((CHUNK, D), jnp.float32),
            pltpu.VMEM((CHUNK, D), jnp.float32),
            pltpu.SemaphoreType.DMA,
            pltpu.SemaphoreType.DMA,
        ],
    )(y, src_r, dst_r, z128)


def kernel(x, edge_index, W1, R1, b1, W2, R2, b2):
    n_edges = edge_index.shape[1]
    gran = NW * CHUNK * IDX_BLK
    e_pad = ((n_edges + gran - 1) // gran) * gran
    per_tile = e_pad // NW

    src = edge_index[0].astype(jnp.int32)
    dst = edge_index[1].astype(jnp.int32)
    pad_idx = jnp.full((e_pad - n_edges,), N_NODES, dtype=jnp.int32)
    src_r = jnp.concatenate([src, pad_idx]).reshape(NW, per_tile // CHUNK, CHUNK)
    dst_r = jnp.concatenate([dst, pad_idx]).reshape(NW, per_tile // CHUNK, CHUNK)

    x_p = jnp.pad(x, ((0, N_PAD - x.shape[0]), (0, 0)))
    b1_p = b1.reshape(1, D)
    b2_p = b2.reshape(1, D)

    z128 = jnp.zeros((CHUNK, D), jnp.float32)
    ones128 = jnp.ones((CHUNK, D), jnp.float32)

    acc1, cnt = _sc_segsum_counts(x_p, src_r, dst_r, z128, ones128)
    h = _tc_layer(acc1[0], acc1[1], cnt[0], cnt[1], x_p, W1, R1, b1_p)
    acc2 = _sc_segsum(h, src_r, dst_r, z128)
    out = _tc_layer(acc2[0], acc2[1], cnt[0], cnt[1], h, W2, R2, b2_p)
    return out[:N_NODES]


# speedup vs baseline: 1.1657x; 1.1000x over previous
import jax
import jax.numpy as jnp
from jax import lax
from jax.experimental import pallas as pl
from jax.experimental.pallas import tpu as pltpu
from jax.experimental.pallas import tpu_sc as plsc

N_NODES = 10000
D = 128
N_PAD = 10240
NW = 32
CHUNK = 128
IDX_BLK = 16
ROWS_PER_TILE = N_PAD // 16


def _tc_layer_body(a0_ref, a1_ref, c0_ref, c1_ref, x_ref, w_ref, r_ref, b_ref,
                   o_ref):
    c = jnp.maximum(c0_ref[:, 0:1] + c1_ref[:, 0:1], 1.0)
    agg = (a0_ref[...] + a1_ref[...]) / c
    o_ref[...] = jax.nn.relu(
        jnp.dot(agg, w_ref[...], preferred_element_type=jnp.float32)
        + jnp.dot(x_ref[...], r_ref[...], preferred_element_type=jnp.float32)
        + b_ref[...])


def _tc_layer(a0, a1, c0, c1, x, w, r, b, block=1024):
    n = x.shape[0]
    grid = (n // block,)
    return pl.pallas_call(
        _tc_layer_body,
        grid=grid,
        in_specs=[
            pl.BlockSpec((block, D), lambda i: (i, 0)),
            pl.BlockSpec((block, D), lambda i: (i, 0)),
            pl.BlockSpec((block, D), lambda i: (i, 0)),
            pl.BlockSpec((block, D), lambda i: (i, 0)),
            pl.BlockSpec((block, D), lambda i: (i, 0)),
            pl.BlockSpec((D, D), lambda i: (0, 0)),
            pl.BlockSpec((D, D), lambda i: (0, 0)),
            pl.BlockSpec((1, D), lambda i: (0, 0)),
        ],
        out_specs=pl.BlockSpec((block, D), lambda i: (i, 0)),
        out_shape=jax.ShapeDtypeStruct((n, D), jnp.float32),
    )(a0, a1, c0, c1, x, w, r, b)


def _seg_pipeline(y_hbm, acc_sh, sd_hbm, g, ibufs, dbufs, gsem, ssem, isem):
    n_blocks = sd_hbm.shape[1]
    total = n_blocks * IDX_BLK

    def six(c):
        return ibufs[(c // IDX_BLK) % 3].at[c % IDX_BLK]

    def dix(c):
        return ibufs[(c // IDX_BLK) % 3].at[IDX_BLK + c % IDX_BLK]

    ipre = {b: pltpu.async_copy(sd_hbm.at[g, b], ibufs[b % 3], isem)
            for b in range(min(2, n_blocks))}
    ipre[0].wait()
    gd = {0: pltpu.async_copy(y_hbm.at[six(0)], dbufs[0], gsem)}
    scat = {}
    waited = 0
    for c in range(total):
        gd[c].wait()
        scat[c] = pltpu.async_copy(dbufs[c % 2], acc_sh.at[dix(c)],
                                   ssem, add=True)
        nc = c + 1
        if nc < total:
            if nc % IDX_BLK == 0:
                b = nc // IDX_BLK
                ipre[b].wait()
                if b + 1 < n_blocks:
                    ipre[b + 1] = pltpu.async_copy(
                        sd_hbm.at[g, b + 1], ibufs[(b + 1) % 3], isem)
            while waited <= nc - 2:
                scat[waited].wait()
                waited += 1
            gd[nc] = pltpu.async_copy(y_hbm.at[six(nc)], dbufs[nc % 2], gsem)
    for w in range(waited, total):
        scat[w].wait()


def _sc_segsum_counts_body(y_hbm, sd_hbm, z128_hbm, ones_hbm,
                           acc_out, cnt_out,
                           acc_sh, idxA, idxB, idxC, buf0, buf1,
                           gsem, ssem, isem):
    sid = lax.axis_index("s")
    cid = lax.axis_index("c")
    g = cid * 16 + sid
    n_blocks = sd_hbm.shape[1]
    row0 = sid * ROWS_PER_TILE

    def zero_my_slice():
        pltpu.sync_copy(z128_hbm, buf0)
        for t in range(ROWS_PER_TILE // CHUNK):
            pltpu.sync_copy(buf0, acc_sh.at[pl.ds(row0 + t * CHUNK, CHUNK)])

    zero_my_slice()
    pltpu.sync_copy(ones_hbm, buf0)
    plsc.subcore_barrier()

    def step_cnt(blk, carry):
        pltpu.sync_copy(sd_hbm.at[g, blk], idxA)
        descs = [pltpu.async_copy(buf0, acc_sh.at[idxA.at[IDX_BLK + j]],
                                  ssem, add=True)
                 for j in range(IDX_BLK)]
        for d in descs:
            d.wait()
        return carry

    lax.fori_loop(0, n_blocks, step_cnt, 0)
    plsc.subcore_barrier()
    pltpu.sync_copy(acc_sh.at[pl.ds(row0, ROWS_PER_TILE)],
                    cnt_out.at[cid, pl.ds(row0, ROWS_PER_TILE)])

    zero_my_slice()
    plsc.subcore_barrier()
    _seg_pipeline(y_hbm, acc_sh, sd_hbm, g, (idxA, idxB, idxC), (buf0, buf1),
                  gsem, ssem, isem)
    plsc.subcore_barrier()
    pltpu.sync_copy(acc_sh.at[pl.ds(row0, ROWS_PER_TILE)],
                    acc_out.at[cid, pl.ds(row0, ROWS_PER_TILE)])


def _sc_segsum_body(y_hbm, sd_hbm, z128_hbm,
                    acc_out,
                    acc_sh, idxA, idxB, idxC, buf0, buf1, gsem, ssem, isem):
    sid = lax.axis_index("s")
    cid = lax.axis_index("c")
    g = cid * 16 + sid
    row0 = sid * ROWS_PER_TILE

    pltpu.sync_copy(z128_hbm, buf0)
    for t in range(ROWS_PER_TILE // CHUNK):
        pltpu.sync_copy(buf0, acc_sh.at[pl.ds(row0 + t * CHUNK, CHUNK)])
    plsc.subcore_barrier()

    _seg_pipeline(y_hbm, acc_sh, sd_hbm, g, (idxA, idxB, idxC), (buf0, buf1),
                  gsem, ssem, isem)
    plsc.subcore_barrier()
    pltpu.sync_copy(acc_sh.at[pl.ds(row0, ROWS_PER_TILE)],
                    acc_out.at[cid, pl.ds(row0, ROWS_PER_TILE)])


_SC_MESH = plsc.VectorSubcoreMesh(core_axis_name="c", subcore_axis_name="s")

_SC_SCRATCH = [
    pltpu.VMEM_SHARED((N_PAD, D), jnp.float32),
    pltpu.VMEM((2 * IDX_BLK, CHUNK), jnp.int32),
    pltpu.VMEM((2 * IDX_BLK, CHUNK), jnp.int32),
    pltpu.VMEM((2 * IDX_BLK, CHUNK), jnp.int32),
    pltpu.VMEM((CHUNK, D), jnp.float32),
    pltpu.VMEM((CHUNK, D), jnp.float32),
    pltpu.SemaphoreType.DMA,
    pltpu.SemaphoreType.DMA,
    pltpu.SemaphoreType.DMA,
]


def _sc_segsum_counts(y, sd_r, z128, ones128):
    return pl.kernel(
        _sc_segsum_counts_body,
        out_type=[
            jax.ShapeDtypeStruct((2, N_PAD, D), jnp.float32),
            jax.ShapeDtypeStruct((2, N_PAD, D), jnp.float32),
        ],
        mesh=_SC_MESH,
        scratch_types=list(_SC_SCRATCH),
    )(y, sd_r, z128, ones128)


def _sc_segsum(y, sd_r, z128):
    return pl.kernel(
        _sc_segsum_body,
        out_type=jax.ShapeDtypeStruct((2, N_PAD, D), jnp.float32),
        mesh=_SC_MESH,
        scratch_types=list(_SC_SCRATCH),
    )(y, sd_r, z128)


def kernel(x, edge_index, W1, R1, b1, W2, R2, b2):
    n_edges = edge_index.shape[1]
    gran = NW * CHUNK * IDX_BLK
    e_pad = ((n_edges + gran - 1) // gran) * gran
    per_tile = e_pad // NW
    n_blocks = per_tile // (CHUNK * IDX_BLK)

    src = edge_index[0].astype(jnp.int32)
    dst = edge_index[1].astype(jnp.int32)
    pad_idx = jnp.full((e_pad - n_edges,), N_NODES, dtype=jnp.int32)
    src_c = jnp.concatenate([src, pad_idx]).reshape(NW, n_blocks, IDX_BLK, CHUNK)
    dst_c = jnp.concatenate([dst, pad_idx]).reshape(NW, n_blocks, IDX_BLK, CHUNK)
    sd_r = jnp.concatenate([src_c, dst_c], axis=2)

    x_p = jnp.pad(x, ((0, N_PAD - x.shape[0]), (0, 0)))
    b1_p = b1.reshape(1, D)
    b2_p = b2.reshape(1, D)

    z128 = jnp.zeros((CHUNK, D), jnp.float32)
    ones128 = jnp.ones((CHUNK, D), jnp.float32)

    acc1, cnt = _sc_segsum_counts(x_p, sd_r, z128, ones128)
    h = _tc_layer(acc1[0], acc1[1], cnt[0], cnt[1], x_p, W1, R1, b1_p)
    acc2 = _sc_segsum(h, sd_r, z128)
    out = _tc_layer(acc2[0], acc2[1], cnt[0], cnt[1], h, W2, R2, b2_p)
    return out[:N_NODES]


# speedup vs baseline: 1.1685x; 1.0024x over previous
import jax
import jax.numpy as jnp
from jax import lax
from jax.experimental import pallas as pl
from jax.experimental.pallas import tpu as pltpu
from jax.experimental.pallas import tpu_sc as plsc

N_NODES = 10000
D = 128
N_PAD = 10240
NW = 32
CHUNK = 128
IDX_BLK = 16
ROWS_PER_TILE = N_PAD // 16


def _tc_layer_body(a0_ref, a1_ref, c0_ref, c1_ref, x_ref, w_ref, r_ref, b_ref,
                   o_ref):
    c = jnp.maximum(c0_ref[:, 0:1] + c1_ref[:, 0:1], 1.0)
    agg = (a0_ref[...] + a1_ref[...]) / c
    o_ref[...] = jax.nn.relu(
        jnp.dot(agg, w_ref[...], preferred_element_type=jnp.float32)
        + jnp.dot(x_ref[...], r_ref[...], preferred_element_type=jnp.float32)
        + b_ref[...])


def _tc_layer(a0, a1, c0, c1, x, w, r, b, block=1024):
    n = x.shape[0]
    grid = (n // block,)
    return pl.pallas_call(
        _tc_layer_body,
        grid=grid,
        in_specs=[
            pl.BlockSpec((block, D), lambda i: (i, 0)),
            pl.BlockSpec((block, D), lambda i: (i, 0)),
            pl.BlockSpec((block, D), lambda i: (i, 0)),
            pl.BlockSpec((block, D), lambda i: (i, 0)),
            pl.BlockSpec((block, D), lambda i: (i, 0)),
            pl.BlockSpec((D, D), lambda i: (0, 0)),
            pl.BlockSpec((D, D), lambda i: (0, 0)),
            pl.BlockSpec((1, D), lambda i: (0, 0)),
        ],
        out_specs=pl.BlockSpec((block, D), lambda i: (i, 0)),
        out_shape=jax.ShapeDtypeStruct((n, D), jnp.float32),
    )(a0, a1, c0, c1, x, w, r, b)


def _seg_pipeline(y_hbm, acc_sh, sd_hbm, g, ibufs, dbufs, gsem, ssem, isem):
    n_blocks = sd_hbm.shape[1]
    total = n_blocks * IDX_BLK

    def six(c):
        return ibufs[(c // IDX_BLK) % 3].at[c % IDX_BLK]

    def dix(c):
        return ibufs[(c // IDX_BLK) % 3].at[IDX_BLK + c % IDX_BLK]

    ipre = {b: pltpu.async_copy(sd_hbm.at[g, b], ibufs[b % 3], isem)
            for b in range(min(2, n_blocks))}
    ipre[0].wait()
    gd = {0: pltpu.async_copy(y_hbm.at[six(0)], dbufs[0], gsem)}
    scat = {}
    waited = 0
    for c in range(total):
        gd[c].wait()
        scat[c] = pltpu.async_copy(dbufs[c % 2], acc_sh.at[dix(c)],
                                   ssem, add=True)
        nc = c + 1
        if nc < total:
            if nc % IDX_BLK == 0:
                b = nc // IDX_BLK
                ipre[b].wait()
                if b + 1 < n_blocks:
                    ipre[b + 1] = pltpu.async_copy(
                        sd_hbm.at[g, b + 1], ibufs[(b + 1) % 3], isem)
            while waited <= nc - 2:
                scat[waited].wait()
                waited += 1
            gd[nc] = pltpu.async_copy(y_hbm.at[six(nc)], dbufs[nc % 2], gsem)
    for w in range(waited, total):
        scat[w].wait()


def _sc_segsum_counts_body(y_hbm, sd_hbm, z128_hbm, ones_hbm,
                           acc_out, cnt_out,
                           acc_sh, idxA, idxB, idxC, buf0, buf1,
                           gsem, ssem, isem):
    sid = lax.axis_index("s")
    cid = lax.axis_index("c")
    g = cid * 16 + sid
    n_blocks = sd_hbm.shape[1]
    row0 = sid * ROWS_PER_TILE

    def zero_my_slice():
        pltpu.sync_copy(z128_hbm, buf0)
        zs = [pltpu.async_copy(buf0, acc_sh.at[pl.ds(row0 + t * CHUNK, CHUNK)],
                               gsem)
              for t in range(ROWS_PER_TILE // CHUNK)]
        for z in zs:
            z.wait()

    zero_my_slice()
    pltpu.sync_copy(ones_hbm, buf0)
    plsc.subcore_barrier()

    ibufs = (idxA, idxB, idxC)
    ipre = {b: pltpu.async_copy(sd_hbm.at[g, b], ibufs[b % 3], isem)
            for b in range(min(2, n_blocks))}
    prev = None
    for b in range(n_blocks):
        ipre[b].wait()
        descs = [pltpu.async_copy(buf0, acc_sh.at[ibufs[b % 3].at[IDX_BLK + j]],
                                  ssem, add=True)
                 for j in range(IDX_BLK)]
        if prev is not None:
            for d in prev:
                d.wait()
        if b + 2 < n_blocks:
            ipre[b + 2] = pltpu.async_copy(sd_hbm.at[g, b + 2],
                                           ibufs[(b + 2) % 3], isem)
        prev = descs
    for d in prev:
        d.wait()
    plsc.subcore_barrier()
    pltpu.sync_copy(acc_sh.at[pl.ds(row0, ROWS_PER_TILE)],
                    cnt_out.at[cid, pl.ds(row0, ROWS_PER_TILE)])

    zero_my_slice()
    plsc.subcore_barrier()
    _seg_pipeline(y_hbm, acc_sh, sd_hbm, g, ibufs, (buf0, buf1),
                  gsem, ssem, isem)
    plsc.subcore_barrier()
    pltpu.sync_copy(acc_sh.at[pl.ds(row0, ROWS_PER_TILE)],
                    acc_out.at[cid, pl.ds(row0, ROWS_PER_TILE)])


def _sc_segsum_body(y_hbm, sd_hbm, z128_hbm,
                    acc_out,
                    acc_sh, idxA, idxB, idxC, buf0, buf1, gsem, ssem, isem):
    sid = lax.axis_index("s")
    cid = lax.axis_index("c")
    g = cid * 16 + sid
    row0 = sid * ROWS_PER_TILE

    pltpu.sync_copy(z128_hbm, buf0)
    zs = [pltpu.async_copy(buf0, acc_sh.at[pl.ds(row0 + t * CHUNK, CHUNK)],
                           gsem)
          for t in range(ROWS_PER_TILE // CHUNK)]
    for z in zs:
        z.wait()
    plsc.subcore_barrier()

    _seg_pipeline(y_hbm, acc_sh, sd_hbm, g, (idxA, idxB, idxC), (buf0, buf1),
                  gsem, ssem, isem)
    plsc.subcore_barrier()
    pltpu.sync_copy(acc_sh.at[pl.ds(row0, ROWS_PER_TILE)],
                    acc_out.at[cid, pl.ds(row0, ROWS_PER_TILE)])


_SC_MESH = plsc.VectorSubcoreMesh(core_axis_name="c", subcore_axis_name="s")

_SC_SCRATCH = [
    pltpu.VMEM_SHARED((N_PAD, D), jnp.float32),
    pltpu.VMEM((2 * IDX_BLK, CHUNK), jnp.int32),
    pltpu.VMEM((2 * IDX_BLK, CHUNK), jnp.int32),
    pltpu.VMEM((2 * IDX_BLK, CHUNK), jnp.int32),
    pltpu.VMEM((CHUNK, D), jnp.float32),
    pltpu.VMEM((CHUNK, D), jnp.float32),
    pltpu.SemaphoreType.DMA,
    pltpu.SemaphoreType.DMA,
    pltpu.SemaphoreType.DMA,
]


def _sc_segsum_counts(y, sd_r, z128, ones128):
    return pl.kernel(
        _sc_segsum_counts_body,
        out_type=[
            jax.ShapeDtypeStruct((2, N_PAD, D), jnp.float32),
            jax.ShapeDtypeStruct((2, N_PAD, D), jnp.float32),
        ],
        mesh=_SC_MESH,
        scratch_types=list(_SC_SCRATCH),
    )(y, sd_r, z128, ones128)


def _sc_segsum(y, sd_r, z128):
    return pl.kernel(
        _sc_segsum_body,
        out_type=jax.ShapeDtypeStruct((2, N_PAD, D), jnp.float32),
        mesh=_SC_MESH,
        scratch_types=list(_SC_SCRATCH),
    )(y, sd_r, z128)


def kernel(x, edge_index, W1, R1, b1, W2, R2, b2):
    n_edges = edge_index.shape[1]
    gran = NW * CHUNK * IDX_BLK
    e_pad = ((n_edges + gran - 1) // gran) * gran
    per_tile = e_pad // NW
    n_blocks = per_tile // (CHUNK * IDX_BLK)

    src = edge_index[0].astype(jnp.int32)
    dst = edge_index[1].astype(jnp.int32)
    pad_idx = jnp.full((e_pad - n_edges,), N_NODES, dtype=jnp.int32)
    src_c = jnp.concatenate([src, pad_idx]).reshape(NW, n_blocks, IDX_BLK, CHUNK)
    dst_c = jnp.concatenate([dst, pad_idx]).reshape(NW, n_blocks, IDX_BLK, CHUNK)
    sd_r = jnp.concatenate([src_c, dst_c], axis=2)

    x_p = jnp.pad(x, ((0, N_PAD - x.shape[0]), (0, 0)))
    b1_p = b1.reshape(1, D)
    b2_p = b2.reshape(1, D)

    z128 = jnp.zeros((CHUNK, D), jnp.float32)
    ones128 = jnp.ones((CHUNK, D), jnp.float32)

    acc1, cnt = _sc_segsum_counts(x_p, sd_r, z128, ones128)
    h = _tc_layer(acc1[0], acc1[1], cnt[0], cnt[1], x_p, W1, R1, b1_p)
    acc2 = _sc_segsum(h, sd_r, z128)
    out = _tc_layer(acc2[0], acc2[1], cnt[0], cnt[1], h, W2, R2, b2_p)
    return out[:N_NODES]
